# Initial kernel scaffold; baseline (speedup 1.0000x reference)
import jax
import jax.numpy as jnp
from jax.experimental import pallas as pl


def kernel(x, edge_attr, params, edge_index, batch):
    raise NotImplementedError("write your pallas kernel here")


# speedup vs baseline: 2.7158x; 2.7158x over previous
import functools

import jax
import jax.numpy as jnp
from jax import lax
from jax.experimental import pallas as pl
from jax.experimental.pallas import tpu as pltpu
from jax.experimental.pallas import tpu_sc as plsc

_N = 10000
_E = 320000
_D = 128
_G = 16
_NC = 2
_NS = 16
_NW = _NC * _NS
_EPW = _E // _NW
_C = 80
_NCH = _EPW // _C
_Q = 624
_TAIL = _N - _NS * _Q
_ZR = 128


def _bn(h, g, b, eps=1e-5):
    mu = jnp.mean(h, axis=0, keepdims=True)
    var = jnp.mean((h - mu) ** 2, axis=0, keepdims=True)
    return (h - mu) / jnp.sqrt(var + eps) * g + b


def _atom_body(x_ref, w1_ref, b1_ref, g1_ref, be1_ref, w2_ref, b2_ref,
               vn_ref, out_ref):
    h = jnp.dot(x_ref[...], w1_ref[...], preferred_element_type=jnp.float32)
    h = jnp.maximum(_bn(h + b1_ref[...], g1_ref[...], be1_ref[...]), 0.0)
    h = jnp.dot(h, w2_ref[...], preferred_element_type=jnp.float32)
    out_ref[...] = h + b2_ref[...] + vn_ref[...]


def _atom_call(x, ap, vn):
    return pl.pallas_call(
        _atom_body,
        out_shape=jax.ShapeDtypeStruct((_N, _D), jnp.float32),
    )(x, ap['W1'], ap['b1'].reshape(1, _D), ap['g1'].reshape(1, _D),
      ap['be1'].reshape(1, _D), ap['W2'], ap['b2'].reshape(1, _D), vn)


def _edge_body(ea_ref, wb_ref, bb_ref, out_ref):
    out_ref[...] = jnp.dot(ea_ref[...], wb_ref[...],
                           preferred_element_type=jnp.float32) + bb_ref[...]


def _edge_call(ea, wb, bb):
    blk = 2560
    return pl.pallas_call(
        _edge_body,
        grid=(_E // blk,),
        in_specs=[
            pl.BlockSpec((blk, 16), lambda i: (i, 0)),
            pl.BlockSpec((16, _D), lambda i: (0, 0)),
            pl.BlockSpec((1, _D), lambda i: (0, 0)),
        ],
        out_specs=pl.BlockSpec((blk, _D), lambda i: (i, 0)),
        out_shape=jax.ShapeDtypeStruct((_E, _D), jnp.float32),
    )(ea, wb, bb.reshape(1, _D))


_R = 1000


def _acc(ref, i, contrib):
    ref[...] = jnp.where(i == 0, contrib, ref[...] + contrib)


def _p1_body(hl_ref, a0_ref, a1_ref, eps_ref, wm1_ref, bm1_ref, batch_ref,
             y_ref, cs_ref, cq_ref, pool_ref):
    i = pl.program_id(0)
    z = (1.0 + eps_ref[0, 0]) * hl_ref[...] + a0_ref[...] + a1_ref[...]
    y = jnp.dot(z, wm1_ref[...], preferred_element_type=jnp.float32) + bm1_ref[...]
    y_ref[...] = y
    _acc(cs_ref, i, jnp.sum(y, axis=0, keepdims=True))
    _acc(cq_ref, i, jnp.sum(y * y, axis=0, keepdims=True))
    onehot = (batch_ref[...] ==
              lax.broadcasted_iota(jnp.int32, (1, _G), 1)).astype(jnp.float32)
    _acc(pool_ref, i, lax.dot_general(
        onehot, hl_ref[...], (((0,), (0,)), ((), ())),
        preferred_element_type=jnp.float32,
        precision=lax.Precision.HIGHEST))


def _p1_call(hl, a0, a1, lp, batch):
    c0 = lambda i: (0, 0)
    return pl.pallas_call(
        _p1_body,
        grid=(_N // _R,),
        in_specs=[
            pl.BlockSpec((_R, _D), lambda i: (i, 0)),
            pl.BlockSpec((_R, _D), lambda i: (i, 0)),
            pl.BlockSpec((_R, _D), lambda i: (i, 0)),
            pl.BlockSpec((1, 1), c0),
            pl.BlockSpec((_D, 2 * _D), c0),
            pl.BlockSpec((1, 2 * _D), c0),
            pl.BlockSpec((_R, 1), lambda i: (i, 0)),
        ],
        out_specs=[
            pl.BlockSpec((_R, 2 * _D), lambda i: (i, 0)),
            pl.BlockSpec((1, 2 * _D), c0),
            pl.BlockSpec((1, 2 * _D), c0),
            pl.BlockSpec((_G, _D), c0),
        ],
        out_shape=[
            jax.ShapeDtypeStruct((_N, 2 * _D), jnp.float32),
            jax.ShapeDtypeStruct((1, 2 * _D), jnp.float32),
            jax.ShapeDtypeStruct((1, 2 * _D), jnp.float32),
            jax.ShapeDtypeStruct((_G, _D), jnp.float32),
        ],
    )(hl, a0, a1, lp['eps'].reshape(1, 1), lp['Wm1'],
      lp['bm1'].reshape(1, 2 * _D), batch.reshape(_N, 1))


def _p2_body(y_ref, cs_ref, cq_ref, mg1_ref, mb1_ref, wm2_ref, bm2_ref,
             w_ref, cs2_ref, cq2_ref):
    i = pl.program_id(0)
    mu = cs_ref[...] * (1.0 / _N)
    var = cq_ref[...] * (1.0 / _N) - mu * mu
    t = jnp.maximum((y_ref[...] - mu) / jnp.sqrt(var + 1e-5)
                    * mg1_ref[...] + mb1_ref[...], 0.0)
    w = jnp.dot(t, wm2_ref[...], preferred_element_type=jnp.float32) + bm2_ref[...]
    w_ref[...] = w
    _acc(cs2_ref, i, jnp.sum(w, axis=0, keepdims=True))
    _acc(cq2_ref, i, jnp.sum(w * w, axis=0, keepdims=True))


def _p2_call(y, cs, cq, lp):
    c0 = lambda i: (0, 0)
    return pl.pallas_call(
        _p2_body,
        grid=(_N // _R,),
        in_specs=[
            pl.BlockSpec((_R, 2 * _D), lambda i: (i, 0)),
            pl.BlockSpec((1, 2 * _D), c0),
            pl.BlockSpec((1, 2 * _D), c0),
            pl.BlockSpec((1, 2 * _D), c0),
            pl.BlockSpec((1, 2 * _D), c0),
            pl.BlockSpec((2 * _D, _D), c0),
            pl.BlockSpec((1, _D), c0),
        ],
        out_specs=[
            pl.BlockSpec((_R, _D), lambda i: (i, 0)),
            pl.BlockSpec((1, _D), c0),
            pl.BlockSpec((1, _D), c0),
        ],
        out_shape=[
            jax.ShapeDtypeStruct((_N, _D), jnp.float32),
            jax.ShapeDtypeStruct((1, _D), jnp.float32),
            jax.ShapeDtypeStruct((1, _D), jnp.float32),
        ],
    )(y, cs, cq, lp['mg1'].reshape(1, 2 * _D), lp['mb1'].reshape(1, 2 * _D),
      lp['Wm2'], lp['bm2'].reshape(1, _D))


def _p3_body(w_ref, cs2_ref, cq2_ref, bng_ref, bnb_ref, *rest, last):
    mu = cs2_ref[...] * (1.0 / _N)
    var = cq2_ref[...] * (1.0 / _N) - mu * mu
    o = (w_ref[...] - mu) / jnp.sqrt(var + 1e-5) * bng_ref[...] + bnb_ref[...]
    if last:
        out_ref, = rest
        out_ref[...] = o
        return
    batch_ref, v_ref, out_ref = rest
    onehot = (batch_ref[...] ==
              lax.broadcasted_iota(jnp.int32, (1, _G), 1)).astype(jnp.float32)
    out_ref[...] = jnp.maximum(o, 0.0) + jnp.dot(
        onehot, v_ref[...], preferred_element_type=jnp.float32,
        precision=lax.Precision.HIGHEST)


def _p3_call(w, cs2, cq2, lp, batch, v):
    last = v is None
    c0 = lambda i: (0, 0)
    in_specs = [
        pl.BlockSpec((_R, _D), lambda i: (i, 0)),
        pl.BlockSpec((1, _D), c0),
        pl.BlockSpec((1, _D), c0),
        pl.BlockSpec((1, _D), c0),
        pl.BlockSpec((1, _D), c0),
    ]
    args = [w, cs2, cq2, lp['bn_g'].reshape(1, _D), lp['bn_b'].reshape(1, _D)]
    if not last:
        in_specs += [pl.BlockSpec((_R, 1), lambda i: (i, 0)),
                     pl.BlockSpec((_G, _D), c0)]
        args += [batch.reshape(_N, 1), v]
    return pl.pallas_call(
        functools.partial(_p3_body, last=last),
        grid=(_N // _R,),
        in_specs=in_specs,
        out_specs=pl.BlockSpec((_R, _D), lambda i: (i, 0)),
        out_shape=jax.ShapeDtypeStruct((_N, _D), jnp.float32),
    )(*args)


def _vn_body(pool_ref, vn_ref, w1_ref, b1_ref, g1_ref, be1_ref,
             w2_ref, b2_ref, g2_ref, be2_ref, v_ref):
    pooled = pool_ref[...] + vn_ref[...]
    t = jnp.dot(pooled, w1_ref[...], preferred_element_type=jnp.float32)
    t = jnp.maximum(_bn(t + b1_ref[...], g1_ref[...], be1_ref[...]), 0.0)
    t = jnp.dot(t, w2_ref[...], preferred_element_type=jnp.float32)
    v_ref[...] = jnp.maximum(_bn(t + b2_ref[...], g2_ref[...], be2_ref[...]), 0.0)


def _vn_call(pool, vn, vp):
    return pl.pallas_call(
        _vn_body,
        out_shape=jax.ShapeDtypeStruct((_G, _D), jnp.float32),
    )(pool, vn, vp['W1'], vp['b1'].reshape(1, 2 * _D),
      vp['g1'].reshape(1, 2 * _D), vp['be1'].reshape(1, 2 * _D),
      vp['W2'], vp['b2'].reshape(1, _D),
      vp['g2'].reshape(1, _D), vp['be2'].reshape(1, _D))


def _post_call(hl, aggr, lp, batch, vn, vp):
    y, cs, cq, pool = _p1_call(hl, aggr[0], aggr[1], lp, batch)
    w, cs2, cq2 = _p2_call(y, cs, cq, lp)
    if vp is None:
        return _p3_call(w, cs2, cq2, lp, batch, None)
    v = _vn_call(pool, vn, vp)
    return _p3_call(w, cs2, cq2, lp, batch, v)


def _sc_msgpass(hl, emb, src, dst):
    mesh = plsc.VectorSubcoreMesh(core_axis_name="c", subcore_axis_name="s",
                                  num_cores=_NC, num_subcores=_NS)

    @functools.partial(
        pl.kernel,
        out_type=jax.ShapeDtypeStruct((_NC, _N, _D), jnp.float32),
        mesh=mesh,
        scratch_types=[
            pltpu.VMEM((_C,), jnp.int32),
            pltpu.VMEM((_C,), jnp.int32),
            pltpu.VMEM((_C, _D), jnp.float32),
            pltpu.VMEM((_C, _D), jnp.float32),
            pltpu.VMEM((_ZR, _D), jnp.float32),
            pltpu.VMEM_SHARED((_N, _D), jnp.float32),
            pltpu.SemaphoreType.DMA,
        ],
    )
    def body(hl_hbm, emb_hbm, src_hbm, dst_hbm, out_hbm,
             sidx, didx, rows, embv, zbuf, aggr, sem):
        c = lax.axis_index("c")
        s = lax.axis_index("s")
        wid = s * _NC + c

        def zrow(i, carry):
            for j in range(_D // 16):
                zbuf[i, pl.ds(j * 16, 16)] = jnp.zeros((16,), jnp.float32)
            return carry
        lax.fori_loop(0, _ZR, zrow, 0)
        for k in range(_Q // _ZR):
            pltpu.sync_copy(zbuf, aggr.at[pl.ds(s * _Q + k * _ZR, _ZR)])
        rem = _Q % _ZR
        if rem:
            pltpu.sync_copy(zbuf.at[pl.ds(0, rem)],
                            aggr.at[pl.ds(s * _Q + _Q - rem, rem)])

        @pl.when(s == _NS - 1)
        def _():
            pltpu.sync_copy(zbuf.at[pl.ds(0, _TAIL)],
                            aggr.at[pl.ds(_NS * _Q, _TAIL)])
        plsc.subcore_barrier()

        def chunk(ci, carry):
            base = wid * _EPW + ci * _C
            pltpu.sync_copy(src_hbm.at[pl.ds(base, _C)], sidx)
            pltpu.sync_copy(dst_hbm.at[pl.ds(base, _C)], didx)
            gather = pltpu.async_copy(hl_hbm.at[sidx], rows, sem)
            pltpu.sync_copy(emb_hbm.at[pl.ds(base, _C)], embv)
            gather.wait()

            def crow(i, cc):
                for j in range(_D // 16):
                    sl = pl.ds(j * 16, 16)
                    rows[i, sl] = jnp.maximum(rows[i, sl] + embv[i, sl], 0.0)
                return cc
            lax.fori_loop(0, _C, crow, 0)
            pltpu.sync_copy(rows, aggr.at[didx], add=True)
            return carry
        lax.fori_loop(0, _NCH, chunk, 0)
        plsc.subcore_barrier()
        pltpu.sync_copy(aggr.at[pl.ds(s * _Q, _Q)],
                        out_hbm.at[c, pl.ds(s * _Q, _Q)])

        @pl.when(s == _NS - 1)
        def _():
            pltpu.sync_copy(aggr.at[pl.ds(_NS * _Q, _TAIL)],
                            out_hbm.at[c, pl.ds(_NS * _Q, _TAIL)])

    return body(hl, emb, src, dst)


def kernel(x, edge_attr, params, edge_index, batch):
    src = edge_index[0].astype(jnp.int32)
    dst = edge_index[1].astype(jnp.int32)
    vn = params['vn']

    hl = _atom_call(x, params['atom'], vn)
    for l in range(2):
        lp = params['layers'][l]
        emb = _edge_call(edge_attr, lp['Wb'], lp['bb'])
        aggr = _sc_msgpass(hl, emb, src, dst)
        vp = params['vn_mlp'][l] if l == 0 else None
        hl = _post_call(hl, aggr, lp, batch, vn, vp)
    return hl
